# baseline (speedup 1.0000x reference)
import dataclasses

import jax
import jax.numpy as jnp
from jax import lax
from jax.experimental import pallas as pl
from jax.experimental.pallas import tpu as pltpu
from jax.experimental.pallas import tpu_sc as plsc

_EMB = 32
_NUM_CORES = 2
_NUM_SUBCORES = 16
_LANES = 16
_BT = 128


def _sc_params():
    cp = pltpu.CompilerParams(use_tc_tiling_on_sc=False)
    if "needs_layout_passes" in pltpu.CompilerParams.__dataclass_fields__:
        cp = dataclasses.replace(cp, needs_layout_passes=False)
    return cp


def kernel(inputs, table):
    n_rows, n_cols = inputs.shape
    n_workers = _NUM_CORES * _NUM_SUBCORES
    n_btiles = n_rows // _BT
    units = n_cols * n_btiles
    units_per_worker = units // n_workers

    idx_t = inputs.T
    mesh = plsc.VectorSubcoreMesh(
        core_axis_name="core", subcore_axis_name="subcore"
    )

    @pl.kernel(
        out_type=jax.ShapeDtypeStruct(
            (n_cols, _EMB // 8, n_btiles, 8, _BT), table.dtype
        ),
        mesh=mesh,
        scratch_types=[
            pltpu.VMEM((2, _BT), jnp.int32),
            pltpu.VMEM((2, _BT, _EMB), jnp.float32),
            pltpu.VMEM((_EMB // 8, 8, _BT), jnp.float32),
            pltpu.SemaphoreType.DMA,
        ],
        compiler_params=_sc_params(),
    )
    def _gather(table_hbm, idx_hbm, out_hbm, idx_v, rows_v, ct, sem):
        wid = lax.axis_index("subcore") * _NUM_CORES + lax.axis_index("core")
        g0 = wid * units_per_worker
        lane = lax.iota(jnp.int32, _LANES)

        def load_and_fire(g, buf):
            j = g // n_btiles
            u = lax.rem(g, n_btiles)
            pltpu.sync_copy(
                idx_hbm.at[j, pl.ds(u * _BT, _BT)], idx_v.at[buf]
            )
            pltpu.async_copy(
                table_hbm.at[idx_v.at[buf]], rows_v.at[buf], sem
            )

        load_and_fire(g0, 0)

        @pl.loop(0, units_per_worker)
        def _(k):
            b = lax.rem(k, 2)
            g = g0 + k

            @pl.when(k < units_per_worker - 1)
            def _():
                load_and_fire(g + 1, lax.rem(k + 1, 2))

            pltpu.make_async_copy(
                table_hbm.at[idx_v.at[b]], rows_v.at[b], sem
            ).wait()

            rbuf = rows_v.at[b]
            @pl.loop(0, _BT // _LANES)
            def _(lg):
                l0 = lg * _LANES
                rr = lane + l0
                for c in range(_EMB):
                    v = plsc.load_gather(
                        rbuf, [rr, jnp.full((_LANES,), c, jnp.int32)]
                    )
                    ct[c // 8, c % 8, pl.ds(l0, _LANES)] = v

            j = g // n_btiles
            u = lax.rem(g, n_btiles)
            pltpu.sync_copy(ct, out_hbm.at[j, :, u])

    out5 = _gather(table, idx_t)
    return out5.transpose(2, 4, 0, 1, 3).reshape(n_rows, n_cols, _EMB)


# speedup vs baseline: 1.0340x; 1.0340x over previous
import dataclasses

import jax
import jax.numpy as jnp
from jax import lax
from jax.experimental import pallas as pl
from jax.experimental.pallas import tpu as pltpu
from jax.experimental.pallas import tpu_sc as plsc

_EMB = 32
_NUM_CORES = 2
_NUM_SUBCORES = 16
_LANES = 16
_BT = 128


def _sc_params():
    cp = pltpu.CompilerParams(use_tc_tiling_on_sc=False)
    if "needs_layout_passes" in pltpu.CompilerParams.__dataclass_fields__:
        cp = dataclasses.replace(cp, needs_layout_passes=False)
    return cp


def kernel(inputs, table):
    n_rows, n_cols = inputs.shape
    n_workers = _NUM_CORES * _NUM_SUBCORES
    n_btiles = n_rows // _BT
    units = n_cols * n_btiles
    units_per_worker = units // n_workers

    idx_t = inputs.T
    mesh = plsc.VectorSubcoreMesh(
        core_axis_name="core", subcore_axis_name="subcore"
    )

    @pl.kernel(
        out_type=jax.ShapeDtypeStruct(
            (n_cols, _EMB // 8, n_btiles, 8, _BT), table.dtype
        ),
        mesh=mesh,
        scratch_types=[
            pltpu.VMEM((2, _BT), jnp.int32),
            pltpu.VMEM((2, _BT, _EMB), jnp.float32),
            pltpu.VMEM((2, _EMB // 8, 8, _BT), jnp.float32),
            pltpu.SemaphoreType.DMA,
            pltpu.SemaphoreType.DMA,
        ],
        compiler_params=_sc_params(),
    )
    def _gather(table_hbm, idx_hbm, out_hbm, idx_v, rows_v, ct, sem, sem_w):
        wid = lax.axis_index("subcore") * _NUM_CORES + lax.axis_index("core")
        g0 = wid * units_per_worker
        lane = lax.iota(jnp.int32, _LANES)

        def load_and_fire(g, buf):
            j = g // n_btiles
            u = lax.rem(g, n_btiles)
            pltpu.sync_copy(
                idx_hbm.at[j, pl.ds(u * _BT, _BT)], idx_v.at[buf]
            )
            pltpu.async_copy(
                table_hbm.at[idx_v.at[buf]], rows_v.at[buf], sem
            )

        load_and_fire(g0, 0)

        @pl.loop(0, units_per_worker)
        def _(k):
            b = lax.rem(k, 2)
            g = g0 + k
            j = g // n_btiles
            u = lax.rem(g, n_btiles)

            @pl.when(k < units_per_worker - 1)
            def _():
                load_and_fire(g + 1, lax.rem(k + 1, 2))

            pltpu.make_async_copy(
                table_hbm.at[idx_v.at[b]], rows_v.at[b], sem
            ).wait()

            @pl.when(k >= 2)
            def _():
                pltpu.make_async_copy(
                    ct.at[b], out_hbm.at[j, :, u], sem_w
                ).wait()

            rbuf = rows_v.at[b]
            for lg in range(_BT // _LANES):
                rr = lane + lg * _LANES
                for c in range(_EMB):
                    v = plsc.load_gather(
                        rbuf, [rr, jnp.full((_LANES,), c, jnp.int32)]
                    )
                    ct[b, c // 8, c % 8, pl.ds(lg * _LANES, _LANES)] = v

            pltpu.async_copy(ct.at[b], out_hbm.at[j, :, u], sem_w)

        for b in range(2):
            pltpu.make_async_copy(
                ct.at[b], out_hbm.at[0, :, 0], sem_w
            ).wait()

    out5 = _gather(table, idx_t)
    return out5.transpose(2, 4, 0, 1, 3).reshape(n_rows, n_cols, _EMB)


# speedup vs baseline: 1.5802x; 1.5281x over previous
import dataclasses

import jax
import jax.numpy as jnp
from jax import lax
from jax.experimental import pallas as pl
from jax.experimental.pallas import tpu as pltpu
from jax.experimental.pallas import tpu_sc as plsc

_EMB = 32
_NUM_CORES = 2
_NUM_SUBCORES = 16
_LANES = 16
_BT = 128


def _sc_params():
    cp = pltpu.CompilerParams(use_tc_tiling_on_sc=False)
    if "needs_layout_passes" in pltpu.CompilerParams.__dataclass_fields__:
        cp = dataclasses.replace(cp, needs_layout_passes=False)
    return cp


def kernel(inputs, table):
    n_rows, n_cols = inputs.shape
    n_workers = _NUM_CORES * _NUM_SUBCORES
    n_btiles = n_rows // _BT
    units = n_cols * n_btiles
    units_per_worker = units // n_workers

    idx_t = inputs.T
    mesh = plsc.VectorSubcoreMesh(
        core_axis_name="core", subcore_axis_name="subcore"
    )

    @pl.kernel(
        out_type=jax.ShapeDtypeStruct(
            (n_cols, _EMB // 8, n_btiles, 8, _BT), table.dtype
        ),
        mesh=mesh,
        scratch_types=[
            pltpu.VMEM((2, _BT), jnp.int32),
            pltpu.VMEM((2, _BT, _EMB), jnp.float32),
            pltpu.VMEM((2, _EMB, _BT + 1), jnp.float32),
            pltpu.SemaphoreType.DMA,
            pltpu.SemaphoreType.DMA,
        ],
        compiler_params=_sc_params(),
    )
    def _gather(table_hbm, idx_hbm, out_hbm, idx_v, rows_v, ct, sem, sem_w):
        wid = lax.axis_index("subcore") * _NUM_CORES + lax.axis_index("core")
        g0 = wid * units_per_worker
        lane = lax.iota(jnp.int32, _LANES)

        def load_and_fire(g, buf):
            j = g // n_btiles
            u = lax.rem(g, n_btiles)
            pltpu.sync_copy(
                idx_hbm.at[j, pl.ds(u * _BT, _BT)], idx_v.at[buf]
            )
            pltpu.async_copy(
                table_hbm.at[idx_v.at[buf]], rows_v.at[buf], sem
            )

        load_and_fire(g0, 0)

        @pl.loop(0, units_per_worker)
        def _(k):
            b = lax.rem(k, 2)
            g = g0 + k
            j = g // n_btiles
            u = lax.rem(g, n_btiles)

            @pl.when(k < units_per_worker - 1)
            def _():
                load_and_fire(g + 1, lax.rem(k + 1, 2))

            pltpu.make_async_copy(
                table_hbm.at[idx_v.at[b]], rows_v.at[b], sem
            ).wait()

            @pl.when(k >= 2)
            def _():
                for r in range(_EMB // 8):
                    pltpu.make_async_copy(
                        ct.at[b, pl.ds(r * 8, 8), pl.ds(0, _BT)],
                        out_hbm.at[j, r, u],
                        sem_w,
                    ).wait()

            rbuf = rows_v.at[b]
            cbuf = ct.at[b]
            for cg in range(_EMB // _LANES):
                c_vec = lane + cg * _LANES
                for l in range(_BT):
                    v = rbuf[l, pl.ds(cg * _LANES, _LANES)]
                    plsc.store_scatter(
                        cbuf,
                        [c_vec, jnp.full((_LANES,), l, jnp.int32)],
                        v,
                    )

            for r in range(_EMB // 8):
                pltpu.async_copy(
                    ct.at[b, pl.ds(r * 8, 8), pl.ds(0, _BT)],
                    out_hbm.at[j, r, u],
                    sem_w,
                )

        for b in range(2):
            for r in range(_EMB // 8):
                pltpu.make_async_copy(
                    ct.at[b, pl.ds(r * 8, 8), pl.ds(0, _BT)],
                    out_hbm.at[0, r, 0],
                    sem_w,
                ).wait()

    out5 = _gather(table, idx_t)
    return out5.transpose(2, 4, 0, 1, 3).reshape(n_rows, n_cols, _EMB)


# speedup vs baseline: 1.5821x; 1.0013x over previous
import dataclasses

import jax
import jax.numpy as jnp
from jax import lax
from jax.experimental import pallas as pl
from jax.experimental.pallas import tpu as pltpu
from jax.experimental.pallas import tpu_sc as plsc

_EMB = 32
_NUM_CORES = 2
_NUM_SUBCORES = 16
_LANES = 16
_BT = 128


def _sc_params():
    cp = pltpu.CompilerParams(use_tc_tiling_on_sc=False)
    if "needs_layout_passes" in pltpu.CompilerParams.__dataclass_fields__:
        cp = dataclasses.replace(cp, needs_layout_passes=False)
    return cp


def kernel(inputs, table):
    n_rows, n_cols = inputs.shape
    n_workers = _NUM_CORES * _NUM_SUBCORES
    n_btiles = n_rows // _BT
    units = n_cols * n_btiles
    units_per_worker = units // n_workers

    idx_t = inputs.T
    mesh = plsc.VectorSubcoreMesh(
        core_axis_name="core", subcore_axis_name="subcore"
    )

    @pl.kernel(
        out_type=jax.ShapeDtypeStruct(
            (n_cols, _EMB // 8, n_btiles, 8, _BT), table.dtype
        ),
        mesh=mesh,
        scratch_types=[
            pltpu.VMEM((2, _BT), jnp.int32),
            pltpu.VMEM((2, _BT, _EMB), jnp.float32),
            pltpu.VMEM((2, _EMB, _BT + 1), jnp.float32),
            pltpu.SemaphoreType.DMA,
            pltpu.SemaphoreType.DMA,
        ],
        compiler_params=_sc_params(),
    )
    def _gather(table_hbm, idx_hbm, out_hbm, idx_v, rows_v, ct, sem, sem_w):
        wid = lax.axis_index("subcore") * _NUM_CORES + lax.axis_index("core")
        g0 = wid * units_per_worker
        lane = lax.iota(jnp.int32, _LANES)

        def load_and_fire(g, buf):
            j = g // n_btiles
            u = lax.rem(g, n_btiles)
            pltpu.sync_copy(
                idx_hbm.at[j, pl.ds(u * _BT, _BT)], idx_v.at[buf]
            )
            pltpu.async_copy(
                table_hbm.at[idx_v.at[buf]], rows_v.at[buf], sem
            )

        load_and_fire(g0, 0)

        @pl.loop(0, units_per_worker)
        def _(k):
            b = lax.rem(k, 2)
            g = g0 + k
            j = g // n_btiles
            u = lax.rem(g, n_btiles)

            @pl.when(k < units_per_worker - 1)
            def _():
                load_and_fire(g + 1, lax.rem(k + 1, 2))

            pltpu.make_async_copy(
                table_hbm.at[idx_v.at[b]], rows_v.at[b], sem
            ).wait()

            @pl.when(k >= 2)
            def _():
                for r in range(_EMB // 8):
                    pltpu.make_async_copy(
                        ct.at[b, pl.ds(r * 8, 8), pl.ds(0, _BT)],
                        out_hbm.at[j, r, u],
                        sem_w,
                    ).wait()

            rbuf = rows_v.at[b]
            cbuf = ct.at[b]
            c_vecs = [
                lane + cg * _LANES for cg in range(_EMB // _LANES)
            ]
            for l in range(_BT):
                l_vec = jnp.full((_LANES,), l, jnp.int32)
                for cg in range(_EMB // _LANES):
                    v = rbuf[l, pl.ds(cg * _LANES, _LANES)]
                    plsc.store_scatter(cbuf, [c_vecs[cg], l_vec], v)

            for r in range(_EMB // 8):
                pltpu.async_copy(
                    ct.at[b, pl.ds(r * 8, 8), pl.ds(0, _BT)],
                    out_hbm.at[j, r, u],
                    sem_w,
                )

        for b in range(2):
            for r in range(_EMB // 8):
                pltpu.make_async_copy(
                    ct.at[b, pl.ds(r * 8, 8), pl.ds(0, _BT)],
                    out_hbm.at[0, r, 0],
                    sem_w,
                ).wait()

    out5 = _gather(table, idx_t)
    return out5.transpose(2, 4, 0, 1, 3).reshape(n_rows, n_cols, _EMB)


# speedup vs baseline: 1.8044x; 1.1405x over previous
import dataclasses

import jax
import jax.numpy as jnp
from jax import lax
from jax.experimental import pallas as pl
from jax.experimental.pallas import tpu as pltpu
from jax.experimental.pallas import tpu_sc as plsc

_EMB = 32
_NUM_CORES = 2
_NUM_SUBCORES = 16
_LANES = 16
_BT = 128


def _sc_params():
    cp = pltpu.CompilerParams(use_tc_tiling_on_sc=False)
    if "needs_layout_passes" in pltpu.CompilerParams.__dataclass_fields__:
        cp = dataclasses.replace(cp, needs_layout_passes=False)
    return cp


def kernel(inputs, table):
    n_rows, n_cols = inputs.shape
    n_workers = _NUM_CORES * _NUM_SUBCORES
    n_btiles = n_rows // _BT
    units = n_cols * n_btiles
    units_per_worker = units // n_workers

    idx_t = inputs.T
    mesh = plsc.VectorSubcoreMesh(
        core_axis_name="core", subcore_axis_name="subcore"
    )

    @pl.kernel(
        out_type=jax.ShapeDtypeStruct(
            (n_cols, _EMB // 8, n_btiles, 8, _BT), table.dtype
        ),
        mesh=mesh,
        scratch_types=[
            pltpu.VMEM((2, _BT), jnp.int32),
            pltpu.VMEM((2, _BT, _EMB), jnp.float32),
            pltpu.VMEM((2, _EMB, _BT + 1), jnp.float32),
            pltpu.SemaphoreType.DMA,
            pltpu.SemaphoreType.DMA,
            pltpu.SemaphoreType.DMA,
        ],
        compiler_params=_sc_params(),
    )
    def _gather(
        table_hbm, idx_hbm, out_hbm, idx_v, rows_v, ct, sem, sem_w, sem_i
    ):
        wid = lax.axis_index("subcore") * _NUM_CORES + lax.axis_index("core")
        g0 = wid * units_per_worker
        lane = lax.iota(jnp.int32, _LANES)

        def fire_idx(g, buf):
            j = g // n_btiles
            u = lax.rem(g, n_btiles)
            pltpu.async_copy(
                idx_hbm.at[j, pl.ds(u * _BT, _BT)], idx_v.at[buf], sem_i
            )

        def drain_idx(buf):
            pltpu.make_async_copy(
                idx_hbm.at[0, pl.ds(0, _BT)], idx_v.at[buf], sem_i
            ).wait()

        fire_idx(g0, 0)
        drain_idx(0)
        pltpu.async_copy(table_hbm.at[idx_v.at[0]], rows_v.at[0], sem)
        fire_idx(g0 + 1, 1)

        @pl.loop(0, units_per_worker)
        def _(k):
            b = lax.rem(k, 2)
            nb = lax.rem(k + 1, 2)
            g = g0 + k
            j = g // n_btiles
            u = lax.rem(g, n_btiles)

            @pl.when(k < units_per_worker - 1)
            def _():
                drain_idx(nb)
                pltpu.async_copy(
                    table_hbm.at[idx_v.at[nb]], rows_v.at[nb], sem
                )

            pltpu.make_async_copy(
                table_hbm.at[idx_v.at[b]], rows_v.at[b], sem
            ).wait()

            @pl.when(k < units_per_worker - 2)
            def _():
                fire_idx(g + 2, b)

            @pl.when(k >= 2)
            def _():
                for r in range(_EMB // 8):
                    pltpu.make_async_copy(
                        ct.at[b, pl.ds(r * 8, 8), pl.ds(0, _BT)],
                        out_hbm.at[j, r, u],
                        sem_w,
                    ).wait()

            rbuf = rows_v.at[b]
            cbuf = ct.at[b]
            c_vecs = [
                lane + cg * _LANES for cg in range(_EMB // _LANES)
            ]
            for l in range(_BT):
                l_vec = jnp.full((_LANES,), l, jnp.int32)
                for cg in range(_EMB // _LANES):
                    v = rbuf[l, pl.ds(cg * _LANES, _LANES)]
                    plsc.store_scatter(cbuf, [c_vecs[cg], l_vec], v)

            for r in range(_EMB // 8):
                pltpu.async_copy(
                    ct.at[b, pl.ds(r * 8, 8), pl.ds(0, _BT)],
                    out_hbm.at[j, r, u],
                    sem_w,
                )

        for b in range(2):
            for r in range(_EMB // 8):
                pltpu.make_async_copy(
                    ct.at[b, pl.ds(r * 8, 8), pl.ds(0, _BT)],
                    out_hbm.at[0, r, 0],
                    sem_w,
                ).wait()

    out5 = _gather(table, idx_t)
    return out5.transpose(2, 4, 0, 1, 3).reshape(n_rows, n_cols, _EMB)
